# baseline (device time: 57170 ns/iter reference)
import functools

import jax
import jax.numpy as jnp
from jax import lax
from jax.experimental import pallas as pl
from jax.experimental.pallas import tpu as pltpu

N_DEV = 4
SQ = 1024
HQ = 8
DH = 128
D = HQ * DH
WINDOW = 128
SCALE = 0.08838834764831843
BF = jnp.bfloat16


def kernel(x, Wq, K_ext, V_ext, Wo):
    def body(x_ref, wq_ref, k_ref, v_ref, wo_ref, out_ref,
             kb, vb, k_loc, v_loc, ob_all, wq_bf, wo_bf,
             kv_send_sems, kv_recv_sems,
             ag_send_sems, ag_recv_sems):
        my = lax.axis_index("i")

        with jax.named_scope("entry_barrier"):
            barrier_sem = pltpu.get_barrier_semaphore()
            for off in range(1, N_DEV):
                pl.semaphore_signal(
                    barrier_sem, inc=1,
                    device_id=(lax.rem(my + off, N_DEV),),
                    device_id_type=pl.DeviceIdType.MESH,
                )
            pl.semaphore_wait(barrier_sem, N_DEV - 1)

        def kv_rdma(src, dst, ssem):
            return pltpu.make_async_remote_copy(
                src_ref=src.at[pl.ds(640, 384), :, :],
                dst_ref=dst.at[pl.ds(0, 384), :, :],
                send_sem=kv_send_sems.at[ssem],
                recv_sem=kv_recv_sems.at[ssem],
                device_id=(1,),
                device_id_type=pl.DeviceIdType.MESH,
            )

        def ag_rdma(r0, nrows, tgt, ssem, rsem):
            return pltpu.make_async_remote_copy(
                src_ref=ob_all.at[pl.ds(r0, nrows), :],
                dst_ref=ob_all.at[pl.ds(r0, nrows), :],
                send_sem=ag_send_sems.at[ssem],
                recv_sem=ag_recv_sems.at[rsem],
                device_id=(tgt,),
                device_id_type=pl.DeviceIdType.MESH,
            )

        def do_block(row0, nrows, koff, nkeys, q_bf, k_src, v_src):
            with jax.named_scope(f"block_r{row0}"):
                q_idx = row0 + lax.broadcasted_iota(
                    jnp.int32, (nrows, nkeys), 0)
                k_idx = koff + lax.broadcasted_iota(
                    jnp.int32, (nrows, nkeys), 1)
                mask = jnp.abs(q_idx - k_idx) <= WINDOW

                ctx_heads = []
                for h in range(HQ):
                    q_h = q_bf[:, h * DH:(h + 1) * DH]
                    k_h = k_src[:, h, :]
                    v_h = v_src[:, h, :]
                    s = lax.dot_general(
                        q_h, k_h, (((1,), (1,)), ((), ())),
                        preferred_element_type=jnp.float32,
                    ) * SCALE
                    w = jnp.exp(jnp.where(mask, s, -1e9))
                    inv = 1.0 / jnp.sum(w, axis=1, keepdims=True)
                    ctx_heads.append(
                        jnp.dot(w.astype(BF), v_h,
                                preferred_element_type=jnp.float32) * inv)
                ctx = jnp.concatenate(ctx_heads, axis=1).astype(BF)
                out_rows = jnp.dot(ctx, wo_bf[:, :],
                                   preferred_element_type=jnp.float32)
                ob_all[row0:row0 + nrows, :] = out_rows.astype(BF)

        @pl.when(my == 0)
        def _():
            with jax.named_scope("cast_kv0"):
                kb[:, :, :] = k_ref[0, :, :, :].astype(BF)
                vb[:, :, :] = v_ref[0, :, :, :].astype(BF)
            kv_rdma(kb, k_loc, 0).start()
            kv_rdma(vb, v_loc, 1).start()
            with jax.named_scope("cast_w0"):
                wq_bf[:, :] = wq_ref[:, :].astype(BF)
                wo_bf[:, :] = wo_ref[:, :].astype(BF)

            with jax.named_scope("qproj0"):
                q768 = jnp.dot(
                    x_ref[0, 0:768, :].astype(BF), wq_bf[:, :],
                    preferred_element_type=jnp.float32).astype(BF)

            do_block(0, 256, 0, 384, q768[0:256, :],
                     kb.at[pl.ds(0, 384)], vb.at[pl.ds(0, 384)])
            ag_rdma(0, 256, 3, 0, 0).start()
            do_block(256, 256, 128, 512, q768[256:512, :],
                     kb.at[pl.ds(128, 512)], vb.at[pl.ds(128, 512)])
            ag_rdma(256, 256, 3, 1, 1).start()

            with jax.named_scope("kv_send_drain"):
                kv_rdma(kb, k_loc, 0).wait_send()
                kv_rdma(vb, v_loc, 1).wait_send()
            ag_rdma(0, 512, 1, 4, 0).start()

            do_block(512, 128, 384, 384, q768[512:640, :],
                     kb.at[pl.ds(384, 384)], vb.at[pl.ds(384, 384)])
            ag_rdma(512, 128, 3, 2, 2).start()
            do_block(640, 128, 512, 384, q768[640:768, :],
                     kb.at[pl.ds(512, 384)], vb.at[pl.ds(512, 384)])
            ag_rdma(640, 128, 3, 3, 3).start()
            ag_rdma(512, 256, 1, 5, 1).start()

            with jax.named_scope("wait_b3"):
                ag_rdma(768, 256, 1, 5, 4).wait_recv()

            with jax.named_scope("drain0"):
                ag_rdma(0, 256, 3, 0, 0).wait_send()
                ag_rdma(256, 256, 3, 1, 1).wait_send()
                ag_rdma(512, 128, 3, 2, 2).wait_send()
                ag_rdma(640, 128, 3, 3, 3).wait_send()
                ag_rdma(0, 512, 1, 4, 0).wait_send()
                ag_rdma(512, 256, 1, 5, 1).wait_send()

        @pl.when(my == 1)
        def _():
            with jax.named_scope("cast1"):
                k_loc[pl.ds(384, 128), :, :] = k_ref[0, 0:128, :, :].astype(BF)
                v_loc[pl.ds(384, 128), :, :] = v_ref[0, 0:128, :, :].astype(BF)
                wq_bf[:, :] = wq_ref[:, :].astype(BF)
                wo_bf[:, :] = wo_ref[:, :].astype(BF)
            with jax.named_scope("kv_wait"):
                kv_rdma(kb, k_loc, 0).wait_recv()
                kv_rdma(vb, v_loc, 1).wait_recv()

            with jax.named_scope("qproj1"):
                q3 = jnp.dot(
                    x_ref[0, 768:1024, :].astype(BF), wq_bf[:, :],
                    preferred_element_type=jnp.float32).astype(BF)
            do_block(768, 256, 640, 512, q3, k_loc, v_loc)

            ag_rdma(768, 256, 0, 0, 4).start()
            ag_rdma(768, 256, 2, 1, 4).start()
            ag_rdma(768, 256, 3, 2, 4).start()

            with jax.named_scope("wait_b012"):
                ag_rdma(0, 512, 0, 4, 0).wait_recv()
                ag_rdma(512, 256, 0, 5, 1).wait_recv()

            with jax.named_scope("drain1"):
                ag_rdma(768, 256, 0, 0, 4).wait_send()
                ag_rdma(768, 256, 2, 1, 4).wait_send()
                ag_rdma(768, 256, 3, 2, 4).wait_send()

        @pl.when(my == 3)
        def _():
            pieces = [(0, 256, 0), (256, 256, 1), (512, 128, 2), (640, 128, 3)]
            for r0, nr, s in pieces:
                with jax.named_scope(f"fw_wait{s}"):
                    ag_rdma(r0, nr, 0, s, s).wait_recv()
                ag_rdma(r0, nr, 2, s, s).start()
            with jax.named_scope("wait_b3_d3"):
                ag_rdma(768, 256, 1, 2, 4).wait_recv()
            with jax.named_scope("drain3"):
                for r0, nr, s in pieces:
                    ag_rdma(r0, nr, 2, s, s).wait_send()

        @pl.when(my == 2)
        def _():
            with jax.named_scope("wait_all_d2"):
                for r0, nr, s in [(0, 256, 0), (256, 256, 1),
                                  (512, 128, 2), (640, 128, 3)]:
                    ag_rdma(r0, nr, 3, s, s).wait_recv()
                ag_rdma(768, 256, 1, 0, 4).wait_recv()

        with jax.named_scope("emit"):
            out_ref[0, :, :] = ob_all[:, :].astype(jnp.float32)

        @functools.partial(
            pl.run_scoped, sem=pltpu.SemaphoreType.REGULAR
        )
        def _(sem):
            for off in range(1, N_DEV):
                pl.semaphore_signal(
                    sem, inc=1,
                    device_id=(lax.rem(my + off, N_DEV),),
                    device_id_type=pl.DeviceIdType.MESH,
                )
            pl.semaphore_wait(sem, N_DEV - 1)

    return pl.pallas_call(
        body,
        out_shape=jax.ShapeDtypeStruct((1, SQ, D), jnp.float32),
        in_specs=[pl.BlockSpec(memory_space=pltpu.VMEM)] * 5,
        out_specs=pl.BlockSpec(memory_space=pltpu.VMEM),
        scratch_shapes=[
            pltpu.VMEM((SQ, HQ, DH), BF),
            pltpu.VMEM((SQ, HQ, DH), BF),
            pltpu.VMEM((512, HQ, DH), BF),
            pltpu.VMEM((512, HQ, DH), BF),
            pltpu.VMEM((SQ, D), BF),
            pltpu.VMEM((D, D), BF),
            pltpu.VMEM((D, D), BF),
            pltpu.SemaphoreType.DMA((2,)),
            pltpu.SemaphoreType.DMA((2,)),
            pltpu.SemaphoreType.DMA((6,)),
            pltpu.SemaphoreType.DMA((6,)),
        ],
        compiler_params=pltpu.CompilerParams(collective_id=0),
    )(x, Wq, K_ext, V_ext, Wo)


# device time: 53558 ns/iter; 1.0674x vs baseline; 1.0674x over previous
import functools

import jax
import jax.numpy as jnp
from jax import lax
from jax.experimental import pallas as pl
from jax.experimental.pallas import tpu as pltpu

N_DEV = 4
SQ = 1024
HQ = 8
DH = 128
D = HQ * DH
WINDOW = 128
SCALE = 0.08838834764831843
BF = jnp.bfloat16


def kernel(x, Wq, K_ext, V_ext, Wo):
    def body(x_ref, wq_ref, k_ref, v_ref, wo_ref, out_ref,
             kb, vb, k_loc, v_loc, ob_all, wq_bf, wo_bf,
             kv_send_sems, kv_recv_sems,
             ag_send_sems, ag_recv_sems):
        my = lax.axis_index("i")

        with jax.named_scope("entry_barrier"):
            barrier_sem = pltpu.get_barrier_semaphore()
            for off in range(1, N_DEV):
                pl.semaphore_signal(
                    barrier_sem, inc=1,
                    device_id=(lax.rem(my + off, N_DEV),),
                    device_id_type=pl.DeviceIdType.MESH,
                )
            pl.semaphore_wait(barrier_sem, N_DEV - 1)

        def kv_rdma(src, dst, ssem):
            return pltpu.make_async_remote_copy(
                src_ref=src.at[pl.ds(640, 384), :, :],
                dst_ref=dst.at[pl.ds(0, 384), :, :],
                send_sem=kv_send_sems.at[ssem],
                recv_sem=kv_recv_sems.at[ssem],
                device_id=(1,),
                device_id_type=pl.DeviceIdType.MESH,
            )

        def ag_rdma(r0, nrows, tgt, ssem, rsem):
            return pltpu.make_async_remote_copy(
                src_ref=ob_all.at[pl.ds(r0, nrows), :],
                dst_ref=ob_all.at[pl.ds(r0, nrows), :],
                send_sem=ag_send_sems.at[ssem],
                recv_sem=ag_recv_sems.at[rsem],
                device_id=(tgt,),
                device_id_type=pl.DeviceIdType.MESH,
            )

        def do_block(row0, nrows, koff, nkeys, k_src, v_src):
            with jax.named_scope(f"block_r{row0}"):
                q_bf = jnp.dot(
                    x_ref[0, row0:row0 + nrows, :].astype(BF), wq_bf[:, :],
                    preferred_element_type=jnp.float32).astype(BF)
                q_idx = row0 + lax.broadcasted_iota(
                    jnp.int32, (nrows, nkeys), 0)
                k_idx = koff + lax.broadcasted_iota(
                    jnp.int32, (nrows, nkeys), 1)
                mask = jnp.abs(q_idx - k_idx) <= WINDOW

                ctx_heads = []
                for h in range(HQ):
                    q_h = q_bf[:, h * DH:(h + 1) * DH]
                    k_h = k_src[:, h, :]
                    v_h = v_src[:, h, :]
                    s = lax.dot_general(
                        q_h, k_h, (((1,), (1,)), ((), ())),
                        preferred_element_type=jnp.float32,
                    ) * SCALE
                    w = jnp.exp(jnp.where(mask, s, -1e9))
                    inv = 1.0 / jnp.sum(w, axis=1, keepdims=True)
                    ctx_heads.append(
                        jnp.dot(w.astype(BF), v_h,
                                preferred_element_type=jnp.float32) * inv)
                ctx = jnp.concatenate(ctx_heads, axis=1).astype(BF)
                out_rows = jnp.dot(ctx, wo_bf[:, :],
                                   preferred_element_type=jnp.float32)
                ob_all[row0:row0 + nrows, :] = out_rows.astype(BF)

        @pl.when(my == 0)
        def _():
            with jax.named_scope("cast_kv0"):
                kb[:, :, :] = k_ref[0, :, :, :].astype(BF)
                vb[:, :, :] = v_ref[0, :, :, :].astype(BF)
            kv_rdma(kb, k_loc, 0).start()
            kv_rdma(vb, v_loc, 1).start()
            with jax.named_scope("cast_w0"):
                wq_bf[:, :] = wq_ref[:, :].astype(BF)
                wo_bf[:, :] = wo_ref[:, :].astype(BF)

            do_block(0, 256, 0, 384,
                     kb.at[pl.ds(0, 384)], vb.at[pl.ds(0, 384)])
            ag_rdma(0, 256, 3, 0, 0).start()
            do_block(256, 256, 128, 512,
                     kb.at[pl.ds(128, 512)], vb.at[pl.ds(128, 512)])
            ag_rdma(256, 256, 3, 1, 1).start()

            with jax.named_scope("kv_send_drain"):
                kv_rdma(kb, k_loc, 0).wait_send()
                kv_rdma(vb, v_loc, 1).wait_send()
            ag_rdma(0, 512, 1, 4, 0).start()

            do_block(512, 128, 384, 384,
                     kb.at[pl.ds(384, 384)], vb.at[pl.ds(384, 384)])
            ag_rdma(512, 128, 3, 2, 2).start()
            do_block(640, 128, 512, 384,
                     kb.at[pl.ds(512, 384)], vb.at[pl.ds(512, 384)])
            ag_rdma(640, 128, 3, 3, 3).start()
            ag_rdma(512, 256, 1, 5, 1).start()

            with jax.named_scope("wait_b3"):
                ag_rdma(768, 256, 1, 5, 4).wait_recv()

            with jax.named_scope("drain0"):
                ag_rdma(0, 256, 3, 0, 0).wait_send()
                ag_rdma(256, 256, 3, 1, 1).wait_send()
                ag_rdma(512, 128, 3, 2, 2).wait_send()
                ag_rdma(640, 128, 3, 3, 3).wait_send()
                ag_rdma(0, 512, 1, 4, 0).wait_send()
                ag_rdma(512, 256, 1, 5, 1).wait_send()

        @pl.when(my == 1)
        def _():
            with jax.named_scope("cast1"):
                k_loc[pl.ds(384, 128), :, :] = k_ref[0, 0:128, :, :].astype(BF)
                v_loc[pl.ds(384, 128), :, :] = v_ref[0, 0:128, :, :].astype(BF)
                wq_bf[:, :] = wq_ref[:, :].astype(BF)
                wo_bf[:, :] = wo_ref[:, :].astype(BF)
            with jax.named_scope("kv_wait"):
                kv_rdma(kb, k_loc, 0).wait_recv()
                kv_rdma(vb, v_loc, 1).wait_recv()

            do_block(768, 256, 640, 512, k_loc, v_loc)

            ag_rdma(768, 256, 0, 0, 4).start()
            ag_rdma(768, 256, 2, 1, 4).start()
            ag_rdma(768, 256, 3, 2, 4).start()

            with jax.named_scope("wait_b012"):
                ag_rdma(0, 512, 0, 4, 0).wait_recv()
                ag_rdma(512, 256, 0, 5, 1).wait_recv()

            with jax.named_scope("drain1"):
                ag_rdma(768, 256, 0, 0, 4).wait_send()
                ag_rdma(768, 256, 2, 1, 4).wait_send()
                ag_rdma(768, 256, 3, 2, 4).wait_send()

        @pl.when(my == 3)
        def _():
            pieces = [(0, 256, 0), (256, 256, 1), (512, 128, 2), (640, 128, 3)]
            for r0, nr, s in pieces:
                with jax.named_scope(f"fw_wait{s}"):
                    ag_rdma(r0, nr, 0, s, s).wait_recv()
                ag_rdma(r0, nr, 2, s, s).start()
            with jax.named_scope("wait_b3_d3"):
                ag_rdma(768, 256, 1, 2, 4).wait_recv()
            with jax.named_scope("drain3"):
                for r0, nr, s in pieces:
                    ag_rdma(r0, nr, 2, s, s).wait_send()

        @pl.when(my == 2)
        def _():
            with jax.named_scope("wait_all_d2"):
                for r0, nr, s in [(0, 256, 0), (256, 256, 1),
                                  (512, 128, 2), (640, 128, 3)]:
                    ag_rdma(r0, nr, 3, s, s).wait_recv()
                ag_rdma(768, 256, 1, 0, 4).wait_recv()

        with jax.named_scope("emit"):
            out_ref[0, :, :] = ob_all[:, :].astype(jnp.float32)

        @functools.partial(
            pl.run_scoped, sem=pltpu.SemaphoreType.REGULAR
        )
        def _(sem):
            for off in range(1, N_DEV):
                pl.semaphore_signal(
                    sem, inc=1,
                    device_id=(lax.rem(my + off, N_DEV),),
                    device_id_type=pl.DeviceIdType.MESH,
                )
            pl.semaphore_wait(sem, N_DEV - 1)

    return pl.pallas_call(
        body,
        out_shape=jax.ShapeDtypeStruct((1, SQ, D), jnp.float32),
        in_specs=[pl.BlockSpec(memory_space=pltpu.VMEM)] * 5,
        out_specs=pl.BlockSpec(memory_space=pltpu.VMEM),
        scratch_shapes=[
            pltpu.VMEM((SQ, HQ, DH), BF),
            pltpu.VMEM((SQ, HQ, DH), BF),
            pltpu.VMEM((512, HQ, DH), BF),
            pltpu.VMEM((512, HQ, DH), BF),
            pltpu.VMEM((SQ, D), BF),
            pltpu.VMEM((D, D), BF),
            pltpu.VMEM((D, D), BF),
            pltpu.SemaphoreType.DMA((2,)),
            pltpu.SemaphoreType.DMA((2,)),
            pltpu.SemaphoreType.DMA((6,)),
            pltpu.SemaphoreType.DMA((6,)),
        ],
        compiler_params=pltpu.CompilerParams(collective_id=0),
    )(x, Wq, K_ext, V_ext, Wo)
